# Initial kernel scaffold; baseline (speedup 1.0000x reference)
import jax
import jax.numpy as jnp
from jax.experimental import pallas as pl


def kernel(data, PI, org_edge_index, embedding, lin_W, att_i, att_j, att_em_i, att_em_j, gl_bias, bn1_g, bn1_b, conv1_W, conv1_b, conv2_W, conv2_b, bn_out_g, bn_out_b, out_W, out_b):
    raise NotImplementedError("write your pallas kernel here")


# speedup vs baseline: 15.1592x; 15.1592x over previous
import functools

import numpy as np

import jax
import jax.numpy as jnp
from jax import lax
from jax.experimental import pallas as pl
from jax.experimental.pallas import tpu as pltpu
from jax.experimental.pallas import tpu_sc as plsc

B = 32
N = 1000
F = 64
D = 64
K = 20
HID = 16
BN_T = B * N
KPAD = 32

_HIGHEST = lax.Precision.HIGHEST


def _graph_build_body(emb_ref, idx2_ref):
    w = emb_ref[:]
    sq = jnp.sum(w * w, axis=1, keepdims=True)
    nrm_col = jnp.sqrt(jnp.sum(w * w, axis=1))
    nrm_row = jnp.sqrt(sq)
    cos = lax.dot_general(w, w, (((1,), (1,)), ((), ())),
                          preferred_element_type=jnp.float32,
                          precision=_HIGHEST)
    cos = cos / (nrm_row * nrm_col)
    colio = lax.broadcasted_iota(jnp.int32, (N, N), 1)
    vals = cos
    cols = []
    for _ in range(K):
        m = jnp.max(vals, axis=1, keepdims=True)
        cand = jnp.where(vals == m, colio, jnp.int32(N + 1))
        it = jnp.min(cand, axis=1)
        cols.append(it[:, None])
        vals = jnp.where(colio == it[:, None], -jnp.inf, vals)
    rowio = lax.broadcasted_iota(jnp.int32, (N, 1), 0)
    for _ in range(KPAD - K):
        cols.append(rowio)
    idx2_ref[:] = jnp.concatenate(cols, axis=1)


def _graph_build(embedding):
    return pl.pallas_call(
        _graph_build_body,
        out_shape=jax.ShapeDtypeStruct((N, KPAD), jnp.int32),
    )(embedding)


def _dense_pre_body(x_ref, linw_ref, attij_ref, emb_ref, attem_ref,
                    xlin_ref, aiaj_ref):
    x = x_ref[:]
    xlin = lax.dot_general(x, linw_ref[:], (((1,), (0,)), ((), ())),
                           preferred_element_type=jnp.float32,
                           precision=_HIGHEST)
    xlin_ref[:] = xlin
    al = lax.dot_general(xlin, attij_ref[:], (((1,), (0,)), ((), ())),
                         preferred_element_type=jnp.float32,
                         precision=_HIGHEST)
    eal = lax.dot_general(emb_ref[:], attem_ref[:], (((1,), (0,)), ((), ())),
                          preferred_element_type=jnp.float32,
                          precision=_HIGHEST)
    eal2 = jnp.concatenate([eal, eal], axis=0)
    aiaj_ref[:] = al + eal2


def _dense_pre(x2d, lin_W, att_ij, embedding, att_em_ij):
    RB = 2 * N
    grid = (BN_T // RB,)
    return pl.pallas_call(
        _dense_pre_body,
        grid=grid,
        in_specs=[
            pl.BlockSpec((RB, F), lambda i: (i, 0)),
            pl.BlockSpec((F, D), lambda i: (0, 0)),
            pl.BlockSpec((D, 2), lambda i: (0, 0)),
            pl.BlockSpec((N, F), lambda i: (0, 0)),
            pl.BlockSpec((F, 2), lambda i: (0, 0)),
        ],
        out_specs=[
            pl.BlockSpec((RB, D), lambda i: (i, 0)),
            pl.BlockSpec((RB, 2), lambda i: (i, 0)),
        ],
        out_shape=[
            jax.ShapeDtypeStruct((BN_T, D), jnp.float32),
            jax.ShapeDtypeStruct((BN_T, 2), jnp.float32),
        ],
    )(x2d, lin_W, att_ij, embedding, att_em_ij)


def _cnn1_body(pat_ref, w1_ref, b1_ref, out_ref):
    f = lax.dot_general(pat_ref[:], w1_ref[:], (((1,), (0,)), ((), ())),
                        preferred_element_type=jnp.float32,
                        precision=_HIGHEST) + b1_ref[:]
    out_ref[:] = jnp.maximum(f, 0.0)


def _cnn1(pat1, w1r, b1):
    return pl.pallas_call(
        _cnn1_body,
        out_shape=jax.ShapeDtypeStruct((B * 256, HID), jnp.float32),
    )(pat1, w1r, b1)


def _cnn2_body(*refs):
    arrs = refs[:16]
    wrefs = refs[16:20]
    b2_ref = refs[20]
    out_ref = refs[21]
    acc = b2_ref[:]
    for q in range(4):
        pooled = jnp.maximum(
            jnp.maximum(arrs[q * 4 + 0][:], arrs[q * 4 + 1][:]),
            jnp.maximum(arrs[q * 4 + 2][:], arrs[q * 4 + 3][:]))
        acc = acc + lax.dot_general(pooled, wrefs[q][:],
                                    (((1,), (0,)), ((), ())),
                                    preferred_element_type=jnp.float32,
                                    precision=_HIGHEST)
    out_ref[:] = jnp.maximum(acc, 0.0)


def _cnn2(arrs, w2rs, b2):
    return pl.pallas_call(
        _cnn2_body,
        out_shape=jax.ShapeDtypeStruct((B * 16, D), jnp.float32),
    )(*arrs, *w2rs, b2)


_SC_CHUNK = 125


def _sc_body(xlin_hbm, aiaj_hbm, idx_hbm, out_hbm, xlin_v, ai_v, aj_v, idx_v,
             out_v):
    info = plsc.get_sparse_core_info()
    nc = info.num_cores
    wid = lax.axis_index("s") * nc + lax.axis_index("c")
    pltpu.sync_copy(xlin_hbm.at[pl.ds(wid * (N * D), N * D)], xlin_v)
    pltpu.sync_copy(aiaj_hbm.at[0, wid], ai_v)
    pltpu.sync_copy(aiaj_hbm.at[1, wid], aj_v)
    pltpu.sync_copy(idx_hbm, idx_v)

    kv1 = lax.iota(jnp.int32, 16) + 16

    def node_body(nl, c):
        n = c * _SC_CHUNK + nl
        nv = jnp.full((16,), n, dtype=jnp.int32)
        vi0 = idx_v[pl.ds(n * KPAD, 16)]
        vi1 = idx_v[pl.ds(n * KPAD + 16, 16)]
        av = plsc.load_gather(ai_v, [nv])
        g0 = plsc.load_gather(aj_v, [vi0])
        g1 = plsc.load_gather(aj_v, [vi1])
        a0 = av + g0
        a0 = jnp.where(a0 > 0, a0, 0.2 * a0)
        a0 = jnp.where(vi0 == nv, jnp.float32(-1e9), a0)
        a1 = av + g1
        a1 = jnp.where(a1 > 0, a1, 0.2 * a1)
        inv1 = ((vi1 == nv) & (kv1 < K)) | (kv1 > K)
        a1 = jnp.where(inv1, jnp.float32(-1e9), a1)
        m = jnp.maximum(jnp.max(a0), jnp.max(a1))
        e0 = jnp.exp(a0 - m)
        e1 = jnp.exp(a1 - m)
        s = jnp.sum(e0) + jnp.sum(e1) + jnp.float32(1e-16)
        w0 = e0 / s
        w1 = e1 / s

        acc = [jnp.zeros((16,), jnp.float32) for _ in range(4)]
        for k in range(K + 1):
            if k < 16:
                j = vi0[k]
                wk = w0[k]
            else:
                j = vi1[k - 16]
                wk = w1[k - 16]
            base = j * D
            for d in range(4):
                row = xlin_v[pl.ds(base + d * 16, 16)]
                acc[d] = acc[d] + wk * row
        ob = nl * D
        for d in range(4):
            out_v[pl.ds(ob + d * 16, 16)] = acc[d]
        return c

    def chunk_body(c, _):
        lax.fori_loop(0, _SC_CHUNK, node_body, c)
        pltpu.sync_copy(
            out_v,
            out_hbm.at[pl.ds(wid * (N * D) + c * (_SC_CHUNK * D),
                             _SC_CHUNK * D)])
        return 0

    lax.fori_loop(0, N // _SC_CHUNK, chunk_body, 0)


def _sc_aggregate(xlin_flat, aiaj, idx_flat):
    mesh = plsc.VectorSubcoreMesh(core_axis_name="c", subcore_axis_name="s")
    fn = functools.partial(
        pl.kernel,
        mesh=mesh,
        compiler_params=pltpu.CompilerParams(needs_layout_passes=False),
        out_type=jax.ShapeDtypeStruct((BN_T * D,), jnp.float32),
        scratch_types=[
            pltpu.VMEM((N * D,), jnp.float32),
            pltpu.VMEM((N,), jnp.float32),
            pltpu.VMEM((N,), jnp.float32),
            pltpu.VMEM((N * KPAD,), jnp.int32),
            pltpu.VMEM((_SC_CHUNK * D,), jnp.float32),
        ],
    )(_sc_body)
    return fn(xlin_flat, aiaj, idx_flat)


def _post_body(*refs):
    g_ref = refs[0]
    f2s = refs[1:17]
    (emb_ref, g1_ref, b1_ref, go_ref, bo_ref, ow_ref, ob_ref,
     out_ref) = refs[17:]
    pi = f2s[0][:]
    for t in range(1, 16):
        pi = jnp.maximum(pi, f2s[t][:])
    mu2 = jnp.mean(pi, axis=0, keepdims=True)
    va2 = jnp.mean((pi - mu2) * (pi - mu2), axis=0, keepdims=True)
    pi = (pi - mu2) / jnp.sqrt(va2 + 1e-5) * g1_ref[:] + b1_ref[:]
    g = g_ref[:]
    mu = jnp.mean(g, axis=0, keepdims=True)
    va = jnp.mean((g - mu) * (g - mu), axis=0, keepdims=True)
    out_t = (g - mu) / jnp.sqrt(va + 1e-5) * g1_ref[:] + b1_ref[:]
    pi_t = jnp.broadcast_to(pi[None], (N, B, D)).reshape(BN_T, D)
    emb_t = jnp.broadcast_to(emb_ref[:][None], (B, N, D)).reshape(BN_T, D)
    z = jnp.maximum(out_t * pi_t, 0.0) * emb_t
    mu3 = jnp.mean(z, axis=0, keepdims=True)
    va3 = jnp.mean((z - mu3) * (z - mu3), axis=0, keepdims=True)
    z = (z - mu3) / jnp.sqrt(va3 + 1e-5) * go_ref[:] + bo_ref[:]
    z = jnp.maximum(z, 0.0)
    out_ref[:] = jnp.sum(z * ow_ref[:], axis=1, keepdims=True) + ob_ref[:]


def _post(gout, f2s, embedding, bn1_g, bn1_b, bn_out_g, bn_out_b, ow, ob):
    return pl.pallas_call(
        _post_body,
        out_shape=jax.ShapeDtypeStruct((BN_T, 1), jnp.float32),
    )(gout, *f2s, embedding, bn1_g, bn1_b, bn_out_g, bn_out_b, ow, ob)


def kernel(data, PI, org_edge_index, embedding, lin_W, att_i, att_j,
           att_em_i, att_em_j, gl_bias, bn1_g, bn1_b, conv1_W, conv1_b,
           conv2_W, conv2_b, bn_out_g, bn_out_b, out_W, out_b):
    x2d = data.reshape(BN_T, F)
    att_ij = jnp.stack([att_i, att_j], axis=1)
    att_em_ij = jnp.stack([att_em_i, att_em_j], axis=1)

    idx2 = _graph_build(embedding)
    xlin, aiaj = _dense_pre(x2d, lin_W, att_ij, embedding, att_em_ij)

    pat1 = PI.reshape(B, 16, 2, 16, 2).transpose(0, 1, 3, 2, 4).reshape(-1, 4)
    w1r = conv1_W.reshape(HID, 4).T
    f = _cnn1(pat1, w1r, conv1_b[None, :])
    f4 = f.reshape(B, 16, 16, HID)
    arrs = [
        f4[:, (2 * d2i + pi0)::4, (2 * d2j + pj0)::4, :].reshape(-1, HID)
        for d2i in range(2) for d2j in range(2)
        for pi0 in range(2) for pj0 in range(2)
    ]
    w2rs = [conv2_W[:, :, d2i, d2j].T for d2i in range(2) for d2j in range(2)]
    f2 = _cnn2(arrs, w2rs, conv2_b[None, :])
    f2r = f2.reshape(B, 16, D)
    f2s = [f2r[:, t, :] for t in range(16)]

    aiaj_t = aiaj.T.reshape(2, B, N)
    gout_flat = _sc_aggregate(xlin.reshape(-1), aiaj_t, idx2.reshape(-1))
    gout = gout_flat.reshape(BN_T, D)

    out = _post(gout, f2s, embedding, bn1_g[None, :], bn1_b[None, :],
                bn_out_g[None, :], bn_out_b[None, :], out_W.T, out_b[None, :])
    return out.reshape(B, N)


# speedup vs baseline: 60.5881x; 3.9968x over previous
import functools

import numpy as np

import jax
import jax.numpy as jnp
from jax import lax
from jax.experimental import pallas as pl
from jax.experimental.pallas import tpu as pltpu
from jax.experimental.pallas import tpu_sc as plsc

B = 32
N = 1000
F = 64
D = 64
K = 20
HID = 16
BN_T = B * N
KPAD = 32

_HIGHEST = lax.Precision.HIGHEST


def _graph_build_body(emb_ref, idx2_ref):
    w = emb_ref[:]
    sq = jnp.sum(w * w, axis=1, keepdims=True)
    nrm_col = jnp.sqrt(jnp.sum(w * w, axis=1))
    nrm_row = jnp.sqrt(sq)
    cos = lax.dot_general(w, w, (((1,), (1,)), ((), ())),
                          preferred_element_type=jnp.float32,
                          precision=_HIGHEST)
    cos = cos / (nrm_row * nrm_col)
    colio = lax.broadcasted_iota(jnp.int32, (N, N), 1)
    vals = cos
    cols = []
    for _ in range(K):
        m = jnp.max(vals, axis=1, keepdims=True)
        cand = jnp.where(vals == m, colio, jnp.int32(N + 1))
        it = jnp.min(cand, axis=1)
        cols.append(it[:, None])
        vals = jnp.where(colio == it[:, None], -jnp.inf, vals)
    rowio = lax.broadcasted_iota(jnp.int32, (N, 1), 0)
    for _ in range(KPAD - K):
        cols.append(rowio)
    idx2_ref[:] = jnp.concatenate(cols, axis=1)


def _graph_build(embedding):
    return pl.pallas_call(
        _graph_build_body,
        out_shape=jax.ShapeDtypeStruct((N, KPAD), jnp.int32),
    )(embedding)


def _dense_pre_body(x_ref, linw_ref, attij_ref, emb_ref, attem_ref,
                    xlin_ref, aiaj_ref):
    x = x_ref[:]
    xlin = lax.dot_general(x, linw_ref[:], (((1,), (0,)), ((), ())),
                           preferred_element_type=jnp.float32,
                           precision=_HIGHEST)
    xlin_ref[:] = xlin
    al = lax.dot_general(xlin, attij_ref[:], (((1,), (0,)), ((), ())),
                         preferred_element_type=jnp.float32,
                         precision=_HIGHEST)
    eal = lax.dot_general(emb_ref[:], attem_ref[:], (((1,), (0,)), ((), ())),
                          preferred_element_type=jnp.float32,
                          precision=_HIGHEST)
    eal2 = jnp.concatenate([eal, eal], axis=0)
    aiaj_ref[:] = al + eal2


def _dense_pre(x2d, lin_W, att_ij, embedding, att_em_ij):
    RB = 2 * N
    grid = (BN_T // RB,)
    return pl.pallas_call(
        _dense_pre_body,
        grid=grid,
        in_specs=[
            pl.BlockSpec((RB, F), lambda i: (i, 0)),
            pl.BlockSpec((F, D), lambda i: (0, 0)),
            pl.BlockSpec((D, 2), lambda i: (0, 0)),
            pl.BlockSpec((N, F), lambda i: (0, 0)),
            pl.BlockSpec((F, 2), lambda i: (0, 0)),
        ],
        out_specs=[
            pl.BlockSpec((RB, D), lambda i: (i, 0)),
            pl.BlockSpec((RB, 2), lambda i: (i, 0)),
        ],
        out_shape=[
            jax.ShapeDtypeStruct((BN_T, D), jnp.float32),
            jax.ShapeDtypeStruct((BN_T, 2), jnp.float32),
        ],
    )(x2d, lin_W, att_ij, embedding, att_em_ij)


def _cnn_body(pat_ref, w1_ref, b1_ref, w2a, w2b, w2c, w2d, b2_ref, out_ref):
    f = lax.dot_general(pat_ref[:], w1_ref[:], (((1,), (0,)), ((), ())),
                        preferred_element_type=jnp.float32,
                        precision=_HIGHEST) + b1_ref[:]
    f = jnp.maximum(f, 0.0)
    wrefs = [w2a, w2b, w2c, w2d]
    acc = b2_ref[:]
    for d2i in range(2):
        for d2j in range(2):
            q = d2i * 2 + d2j
            blks = []
            for pi0 in range(2):
                for pj0 in range(2):
                    c = (2 * d2i + pi0) * 4 + (2 * d2j + pj0)
                    blks.append(f[c * (B * 16):(c + 1) * (B * 16)])
            pooled = jnp.maximum(jnp.maximum(blks[0], blks[1]),
                                 jnp.maximum(blks[2], blks[3]))
            acc = acc + lax.dot_general(pooled, wrefs[q][:],
                                        (((1,), (0,)), ((), ())),
                                        preferred_element_type=jnp.float32,
                                        precision=_HIGHEST)
    z = jnp.maximum(acc, 0.0)
    out_ref[:] = jnp.max(z.reshape(B, 16, D), axis=1)


def _cnn(pat1, w1r, b1, w2rs, b2):
    return pl.pallas_call(
        _cnn_body,
        out_shape=jax.ShapeDtypeStruct((B, D), jnp.float32),
    )(pat1, w1r, b1, *w2rs, b2)


_SC_CHUNK = 125


def _sc_body(xlin_hbm, aiaj_hbm, idx_hbm, out_hbm, xlin_v, ai_v, aj_v, idx_v,
             out_v):
    info = plsc.get_sparse_core_info()
    nc = info.num_cores
    wid = lax.axis_index("s") * nc + lax.axis_index("c")
    pltpu.sync_copy(xlin_hbm.at[pl.ds(wid * (N * D), N * D)], xlin_v)
    pltpu.sync_copy(aiaj_hbm.at[0, wid], ai_v)
    pltpu.sync_copy(aiaj_hbm.at[1, wid], aj_v)
    pltpu.sync_copy(idx_hbm, idx_v)

    kv1 = lax.iota(jnp.int32, 16) + 16

    def node_body(nl, c):
        n = c * _SC_CHUNK + nl
        nv = jnp.full((16,), n, dtype=jnp.int32)
        vi0 = idx_v[pl.ds(n * KPAD, 16)]
        vi1 = idx_v[pl.ds(n * KPAD + 16, 16)]
        av = plsc.load_gather(ai_v, [nv])
        g0 = plsc.load_gather(aj_v, [vi0])
        g1 = plsc.load_gather(aj_v, [vi1])
        a0 = av + g0
        a0 = jnp.where(a0 > 0, a0, 0.2 * a0)
        a0 = jnp.where(vi0 == nv, jnp.float32(-1e9), a0)
        a1 = av + g1
        a1 = jnp.where(a1 > 0, a1, 0.2 * a1)
        inv1 = ((vi1 == nv) & (kv1 < K)) | (kv1 > K)
        a1 = jnp.where(inv1, jnp.float32(-1e9), a1)
        m = jnp.maximum(jnp.max(a0), jnp.max(a1))
        e0 = jnp.exp(a0 - m)
        e1 = jnp.exp(a1 - m)
        s = jnp.sum(e0) + jnp.sum(e1) + jnp.float32(1e-16)
        w0 = e0 / s
        w1 = e1 / s

        acc = [jnp.zeros((16,), jnp.float32) for _ in range(4)]
        for k in range(K + 1):
            if k < 16:
                j = vi0[k]
                wk = w0[k]
            else:
                j = vi1[k - 16]
                wk = w1[k - 16]
            base = j * D
            for d in range(4):
                row = xlin_v[pl.ds(base + d * 16, 16)]
                acc[d] = acc[d] + wk * row
        ob = nl * D
        for d in range(4):
            out_v[pl.ds(ob + d * 16, 16)] = acc[d]
        return c

    def chunk_body(c, _):
        lax.fori_loop(0, _SC_CHUNK, node_body, c)
        pltpu.sync_copy(
            out_v,
            out_hbm.at[pl.ds(wid * (N * D) + c * (_SC_CHUNK * D),
                             _SC_CHUNK * D)])
        return 0

    lax.fori_loop(0, N // _SC_CHUNK, chunk_body, 0)


def _sc_aggregate(xlin_flat, aiaj, idx_flat):
    mesh = plsc.VectorSubcoreMesh(core_axis_name="c", subcore_axis_name="s")
    fn = functools.partial(
        pl.kernel,
        mesh=mesh,
        compiler_params=pltpu.CompilerParams(needs_layout_passes=False),
        out_type=jax.ShapeDtypeStruct((BN_T * D,), jnp.float32),
        scratch_types=[
            pltpu.VMEM((N * D,), jnp.float32),
            pltpu.VMEM((N,), jnp.float32),
            pltpu.VMEM((N,), jnp.float32),
            pltpu.VMEM((N * KPAD,), jnp.int32),
            pltpu.VMEM((_SC_CHUNK * D,), jnp.float32),
        ],
    )(_sc_body)
    return fn(xlin_flat, aiaj, idx_flat)


def _post_body(*refs):
    (g_ref, pi_ref, emb_ref, g1_ref, b1_ref, go_ref, bo_ref, ow_ref, ob_ref,
     out_ref) = refs
    pi = pi_ref[:]
    mu2 = jnp.mean(pi, axis=0, keepdims=True)
    va2 = jnp.mean((pi - mu2) * (pi - mu2), axis=0, keepdims=True)
    pi = (pi - mu2) / jnp.sqrt(va2 + 1e-5) * g1_ref[:] + b1_ref[:]
    g = g_ref[:]
    mu = jnp.mean(g, axis=0, keepdims=True)
    va = jnp.mean((g - mu) * (g - mu), axis=0, keepdims=True)
    out_t = (g - mu) / jnp.sqrt(va + 1e-5) * g1_ref[:] + b1_ref[:]
    pi_t = jnp.broadcast_to(pi[None], (N, B, D)).reshape(BN_T, D)
    emb_t = jnp.broadcast_to(emb_ref[:][None], (B, N, D)).reshape(BN_T, D)
    z = jnp.maximum(out_t * pi_t, 0.0) * emb_t
    mu3 = jnp.mean(z, axis=0, keepdims=True)
    va3 = jnp.mean((z - mu3) * (z - mu3), axis=0, keepdims=True)
    z = (z - mu3) / jnp.sqrt(va3 + 1e-5) * go_ref[:] + bo_ref[:]
    z = jnp.maximum(z, 0.0)
    out_ref[:] = jnp.sum(z * ow_ref[:], axis=1, keepdims=True) + ob_ref[:]


def _post(gout, pi, embedding, bn1_g, bn1_b, bn_out_g, bn_out_b, ow, ob):
    return pl.pallas_call(
        _post_body,
        out_shape=jax.ShapeDtypeStruct((BN_T, 1), jnp.float32),
    )(gout, pi, embedding, bn1_g, bn1_b, bn_out_g, bn_out_b, ow, ob)


def kernel(data, PI, org_edge_index, embedding, lin_W, att_i, att_j,
           att_em_i, att_em_j, gl_bias, bn1_g, bn1_b, conv1_W, conv1_b,
           conv2_W, conv2_b, bn_out_g, bn_out_b, out_W, out_b):
    x2d = data.reshape(BN_T, F)
    att_ij = jnp.stack([att_i, att_j], axis=1)
    att_em_ij = jnp.stack([att_em_i, att_em_j], axis=1)

    idx2 = _graph_build(embedding)
    xlin, aiaj = _dense_pre(x2d, lin_W, att_ij, embedding, att_em_ij)

    pat1 = (PI.reshape(B, 4, 4, 2, 4, 4, 2)
            .transpose(2, 5, 0, 1, 4, 3, 6).reshape(-1, 4))
    w1r = conv1_W.reshape(HID, 4).T
    w2rs = [conv2_W[:, :, d2i, d2j].T for d2i in range(2) for d2j in range(2)]
    pi_feat = _cnn(pat1, w1r, conv1_b[None, :], w2rs, conv2_b[None, :])

    aiaj_t = aiaj.T.reshape(2, B, N)
    gout_flat = _sc_aggregate(xlin.reshape(-1), aiaj_t, idx2.reshape(-1))
    gout = gout_flat.reshape(BN_T, D)

    out = _post(gout, pi_feat, embedding, bn1_g[None, :], bn1_b[None, :],
                bn_out_g[None, :], bn_out_b[None, :], out_W.T, out_b[None, :])
    return out.reshape(B, N)
